# Initial kernel scaffold; baseline (speedup 1.0000x reference)
import jax
import jax.numpy as jnp
from jax.experimental import pallas as pl


def kernel(x, edge_index, edge_weight, W0, W1, W2):
    raise NotImplementedError("write your pallas kernel here")


# speedup vs baseline: 3.8534x; 3.8534x over previous
import functools

import jax
import jax.numpy as jnp
from jax import lax
from jax.experimental import pallas as pl
from jax.experimental.pallas import tpu as pltpu
from jax.experimental.pallas import tpu_sc as plsc

_N = 10000
_E = 160000
_F = 128
_H1 = 32
_H2 = 16

_CHUNK = 128
_NCHUNKS = _E // _CHUNK
_NC = 2
_NS = 16
_NW = _NC * _NS
_CPW = (_NCHUNKS + _NW - 1) // _NW
_RPT8 = (_N // _NS) // 8 * 8
_TAIL = _N - _NS * _RPT8

_ROWS_BLK = 1000
_DEC_BM = 1024
_DEC_BN = 1024


def _spmm_partials(table, edata, ew3, zeros):
    mesh = plsc.VectorSubcoreMesh(core_axis_name="c", subcore_axis_name="s")

    @functools.partial(
        pl.kernel,
        mesh=mesh,
        compiler_params=pltpu.CompilerParams(use_tc_tiling_on_sc=False),
        out_type=jax.ShapeDtypeStruct((_NC, _N, _H1), jnp.float32),
        scratch_types=[
            pltpu.VMEM((2, _CHUNK), jnp.int32),
            pltpu.VMEM((1, _CHUNK), jnp.float32),
            pltpu.VMEM((_CHUNK, _H1), jnp.float32),
            pltpu.VMEM_SHARED((_N, _H1), jnp.float32),
            pltpu.SemaphoreType.DMA,
        ],
    )
    def k(table_hbm, edata_hbm, ew_hbm, zeros_hbm, out_hbm,
          ebuf, wbuf, rows, acc, sem):
        c = lax.axis_index("c")
        s = lax.axis_index("s")
        wid = s * _NC + c

        pltpu.sync_copy(zeros_hbm.at[pl.ds(s * _RPT8, _RPT8)],
                        acc.at[pl.ds(s * _RPT8, _RPT8)])

        @pl.when(s == 0)
        def _():
            pltpu.sync_copy(zeros_hbm.at[pl.ds(_NS * _RPT8, _TAIL)],
                            acc.at[pl.ds(_NS * _RPT8, _TAIL)])

        plsc.subcore_barrier()

        def chunk_body(i, carry):
            chunk = i * _NW + wid

            @pl.when(chunk < _NCHUNKS)
            def _():
                pltpu.sync_copy(edata_hbm.at[chunk], ebuf)
                pltpu.sync_copy(ew_hbm.at[chunk], wbuf)
                pltpu.async_copy(table_hbm.at[ebuf.at[0]], rows, sem).wait()
                for g in range(_CHUNK // 16):
                    wg = wbuf[0, pl.ds(g * 16, 16)]
                    for t in range(16):
                        e = g * 16 + t
                        wb = lax.gather(
                            wg, jnp.full((16, 1), t, jnp.int32),
                            lax.GatherDimensionNumbers(
                                offset_dims=(), collapsed_slice_dims=(0,),
                                start_index_map=(0,)),
                            slice_sizes=(1,),
                            mode=lax.GatherScatterMode.PROMISE_IN_BOUNDS)
                        rows[e, pl.ds(0, 16)] = rows[e, pl.ds(0, 16)] * wb
                        rows[e, pl.ds(16, 16)] = rows[e, pl.ds(16, 16)] * wb
                pltpu.sync_copy(rows, acc.at[ebuf.at[1]], add=True)

            return carry

        lax.fori_loop(0, _CPW, chunk_body, 0)
        plsc.subcore_barrier()
        pltpu.sync_copy(acc.at[pl.ds(s * _RPT8, _RPT8)],
                        out_hbm.at[c, pl.ds(s * _RPT8, _RPT8)])

        @pl.when(s == 0)
        def _():
            pltpu.sync_copy(acc.at[pl.ds(_NS * _RPT8, _TAIL)],
                            out_hbm.at[c, pl.ds(_NS * _RPT8, _TAIL)])

    return k(table, edata, ew3, zeros)


def _matmul_xw0(x, W0):
    def body(x_ref, w_ref, o_ref):
        o_ref[...] = jnp.dot(x_ref[...], w_ref[...],
                             preferred_element_type=jnp.float32)

    return pl.pallas_call(
        body,
        grid=(_N // _ROWS_BLK,),
        in_specs=[
            pl.BlockSpec((_ROWS_BLK, _F), lambda i: (i, 0)),
            pl.BlockSpec((_F, _H1), lambda i: (0, 0)),
        ],
        out_specs=pl.BlockSpec((_ROWS_BLK, _H1), lambda i: (i, 0)),
        out_shape=jax.ShapeDtypeStruct((_N, _H1), jnp.float32),
    )(x, W0)


def _relu_combine(p):
    def body(p_ref, o_ref):
        o_ref[...] = jnp.maximum(p_ref[0] + p_ref[1], 0.0)

    return pl.pallas_call(
        body,
        grid=(_N // _ROWS_BLK,),
        in_specs=[pl.BlockSpec((_NC, _ROWS_BLK, _H1), lambda i: (0, i, 0))],
        out_specs=pl.BlockSpec((_ROWS_BLK, _H1), lambda i: (i, 0)),
        out_shape=jax.ShapeDtypeStruct((_N, _H1), jnp.float32),
    )(p)


def _z_combine(q, W1, W2, eps):
    def body(q_ref, w1_ref, w2_ref, e_ref, o_ref):
        sblk = q_ref[0] + q_ref[1]
        zm = jnp.dot(sblk, w1_ref[...], preferred_element_type=jnp.float32)
        zl = jnp.dot(sblk, w2_ref[...], preferred_element_type=jnp.float32)
        o_ref[...] = zm + e_ref[...] * jnp.exp(zl)

    return pl.pallas_call(
        body,
        grid=(_N // _ROWS_BLK,),
        in_specs=[
            pl.BlockSpec((_NC, _ROWS_BLK, _H1), lambda i: (0, i, 0)),
            pl.BlockSpec((_H1, _H2), lambda i: (0, 0)),
            pl.BlockSpec((_H1, _H2), lambda i: (0, 0)),
            pl.BlockSpec((_ROWS_BLK, _H2), lambda i: (i, 0)),
        ],
        out_specs=pl.BlockSpec((_ROWS_BLK, _H2), lambda i: (i, 0)),
        out_shape=jax.ShapeDtypeStruct((_N, _H2), jnp.float32),
    )(q, W1, W2, eps)


def _decode(z):
    def body(zi_ref, zj_ref, o_ref):
        o_ref[...] = lax.dot_general(
            zi_ref[...], zj_ref[...],
            (((1,), (1,)), ((), ())),
            preferred_element_type=jnp.float32,
        )

    return pl.pallas_call(
        body,
        grid=(pl.cdiv(_N, _DEC_BM), pl.cdiv(_N, _DEC_BN)),
        in_specs=[
            pl.BlockSpec((_DEC_BM, _H2), lambda i, j: (i, 0)),
            pl.BlockSpec((_DEC_BN, _H2), lambda i, j: (j, 0)),
        ],
        out_specs=pl.BlockSpec((_DEC_BM, _DEC_BN), lambda i, j: (i, j)),
        out_shape=jax.ShapeDtypeStruct((_N, _N), jnp.float32),
    )(z, z)


def kernel(x, edge_index, edge_weight, W0, W1, W2):
    src = edge_index[0]
    dst = edge_index[1]
    edata = jnp.stack(
        [src.reshape(_NCHUNKS, _CHUNK),
         dst.reshape(_NCHUNKS, _CHUNK)],
        axis=1,
    )
    ew3 = edge_weight.reshape(_NCHUNKS, 1, _CHUNK)
    zeros = jnp.zeros((_N, _H1), jnp.float32)

    h0 = _matmul_xw0(x, W0)
    p = _spmm_partials(h0, edata, ew3, zeros)
    hidden1 = _relu_combine(p)
    q = _spmm_partials(hidden1, edata, ew3, zeros)

    eps = jax.random.normal(jax.random.key(42), (_N, _H2), dtype=jnp.float32)
    z = _z_combine(q, W1, W2, eps)
    return _decode(z).reshape(-1)
